# baseline (speedup 1.0000x reference)
import functools

import jax
import jax.numpy as jnp
from jax import lax
from jax.experimental import pallas as pl
from jax.experimental.pallas import tpu as pltpu
from jax.experimental.pallas import tpu_sc as plsc

B = 1024
OTHER_LEN = 50
LANES_LEN = 200
BDRY_LEN = 200
EMB_DIM = 64
TOTAL_LEN = 1 + OTHER_LEN + LANES_LEN + BDRY_LEN
ROW = TOTAL_LEN * EMB_DIM
AOAT_PAD = 64

_NC = 2
_NS = 16
_NW = _NC * _NS
_SCENES_PER_W = B // _NW


def _sc_body(aoat_hbm, w_hbm, out_hbm, aoat_v, w_v, tmpl, sem):
    wid = lax.axis_index("s") * _NC + lax.axis_index("c")
    base = wid * _SCENES_PER_W

    pltpu.sync_copy(w_hbm, w_v)
    pltpu.sync_copy(
        aoat_hbm.at[pl.ds(base * AOAT_PAD, _SCENES_PER_W * AOAT_PAD)], aoat_v)

    nch = EMB_DIM // 16
    w0c = [w_v[pl.ds(0 * EMB_DIM + 16 * c, 16)] for c in range(nch)]
    w1c = [w_v[pl.ds(1 * EMB_DIM + 16 * c, 16)] for c in range(nch)]
    w2c = [w_v[pl.ds(2 * EMB_DIM + 16 * c, 16)] for c in range(nch)]
    w5c = [w_v[pl.ds(5 * EMB_DIM + 16 * c, 16)] for c in range(nch)]
    w11c = [w_v[pl.ds(11 * EMB_DIM + 16 * c, 16)] for c in range(nch)]

    for c in range(nch):
        tmpl[pl.ds(16 * c, 16)] = w0c[c]

    def _fill(lo, hi, chunks):
        def body(r, carry):
            for c in range(nch):
                tmpl[pl.ds(r * EMB_DIM + 16 * c, 16)] = chunks[c]
            return carry
        lax.fori_loop(lo, hi, body, 0)

    _fill(1 + OTHER_LEN, 1 + OTHER_LEN + LANES_LEN, w5c)
    _fill(1 + OTHER_LEN + LANES_LEN, TOTAL_LEN, w11c)

    def scene_body(s, carry):
        for j in range(OTHER_LEN):
            sel = plsc.load_gather(
                aoat_v, [jnp.full((16,), s * AOAT_PAD + j, jnp.int32)])
            m = sel == 1
            for c in range(nch):
                tmpl[pl.ds((1 + j) * EMB_DIM + 16 * c, 16)] = jnp.where(
                    m, w2c[c], w1c[c])
        pltpu.sync_copy(tmpl, out_hbm.at[pl.ds((base + s) * ROW, ROW)])
        return carry

    lax.fori_loop(0, _SCENES_PER_W, scene_body, 0)


@functools.partial(jax.jit, static_argnames=())
def _sc_call(aoat_flat, w_flat):
    mesh = plsc.VectorSubcoreMesh(core_axis_name="c", subcore_axis_name="s")
    f = functools.partial(
        pl.kernel,
        out_type=jax.ShapeDtypeStruct((B * ROW,), jnp.float32),
        mesh=mesh,
        compiler_params=pltpu.CompilerParams(needs_layout_passes=False),
        scratch_types=[
            pltpu.VMEM((_SCENES_PER_W * AOAT_PAD,), jnp.int32),
            pltpu.VMEM((12 * EMB_DIM,), jnp.float32),
            pltpu.VMEM((ROW,), jnp.float32),
            pltpu.SemaphoreType.DMA,
        ],
    )(_sc_body)
    return f(aoat_flat, w_flat)


def kernel(type, all_other_agents_types, lanes_mid, lanes, embedding_weight):
    del type, lanes_mid, lanes
    aoat = all_other_agents_types.astype(jnp.int32)
    aoat_flat = jnp.pad(
        aoat, ((0, 0), (0, AOAT_PAD - OTHER_LEN))).reshape(-1)
    w_flat = embedding_weight.reshape(-1)
    out = _sc_call(aoat_flat, w_flat)
    return out.reshape(B, TOTAL_LEN, EMB_DIM)


# speedup vs baseline: 1.1429x; 1.1429x over previous
import functools

import jax
import jax.numpy as jnp
from jax import lax
from jax.experimental import pallas as pl
from jax.experimental.pallas import tpu as pltpu
from jax.experimental.pallas import tpu_sc as plsc

B = 1024
OTHER_LEN = 50
LANES_LEN = 200
BDRY_LEN = 200
EMB_DIM = 64
TOTAL_LEN = 1 + OTHER_LEN + LANES_LEN + BDRY_LEN
ROW = TOTAL_LEN * EMB_DIM
AOAT_PAD = 64

_NC = 2
_NS = 16
_NW = _NC * _NS
_SCENES_PER_W = B // _NW


def _sc_body(aoat_hbm, w_hbm, out_hbm, aoat_v, w_v, tmpl, sem):
    wid = lax.axis_index("s") * _NC + lax.axis_index("c")
    base = wid * _SCENES_PER_W

    pltpu.sync_copy(w_hbm, w_v)
    pltpu.sync_copy(
        aoat_hbm.at[pl.ds(base * AOAT_PAD, _SCENES_PER_W * AOAT_PAD)], aoat_v)

    nch = EMB_DIM // 16
    w0c = [w_v[pl.ds(0 * EMB_DIM + 16 * c, 16)] for c in range(nch)]
    w1c = [w_v[pl.ds(1 * EMB_DIM + 16 * c, 16)] for c in range(nch)]
    w2c = [w_v[pl.ds(2 * EMB_DIM + 16 * c, 16)] for c in range(nch)]
    w5c = [w_v[pl.ds(5 * EMB_DIM + 16 * c, 16)] for c in range(nch)]
    w11c = [w_v[pl.ds(11 * EMB_DIM + 16 * c, 16)] for c in range(nch)]

    for c in range(nch):
        tmpl[0, pl.ds(16 * c, 16)] = w0c[c]

    def _fill(lo, hi, chunks):
        def body(r, carry):
            for c in range(nch):
                tmpl[r, pl.ds(16 * c, 16)] = chunks[c]
            return carry
        lax.fori_loop(lo, hi, body, 0)

    _fill(1 + OTHER_LEN, 1 + OTHER_LEN + LANES_LEN, w5c)
    _fill(1 + OTHER_LEN + LANES_LEN, TOTAL_LEN, w11c)

    def scene_body(s, carry):
        for j in range(OTHER_LEN):
            sel = plsc.load_gather(
                aoat_v, [jnp.full((16,), s * AOAT_PAD + j, jnp.int32)])
            m = sel == 1
            for c in range(nch):
                tmpl[1 + j, pl.ds(16 * c, 16)] = jnp.where(
                    m, w2c[c], w1c[c])
        pltpu.sync_copy(tmpl, out_hbm.at[base + s])
        return carry

    lax.fori_loop(0, _SCENES_PER_W, scene_body, 0)


@functools.partial(jax.jit, static_argnames=())
def _sc_call(aoat_flat, w_flat):
    mesh = plsc.VectorSubcoreMesh(core_axis_name="c", subcore_axis_name="s")
    f = functools.partial(
        pl.kernel,
        out_type=jax.ShapeDtypeStruct((B, TOTAL_LEN, EMB_DIM), jnp.float32),
        mesh=mesh,
        compiler_params=pltpu.CompilerParams(needs_layout_passes=False),
        scratch_types=[
            pltpu.VMEM((_SCENES_PER_W * AOAT_PAD,), jnp.int32),
            pltpu.VMEM((12 * EMB_DIM,), jnp.float32),
            pltpu.VMEM((TOTAL_LEN, EMB_DIM), jnp.float32),
            pltpu.SemaphoreType.DMA,
        ],
    )(_sc_body)
    return f(aoat_flat, w_flat)


def kernel(type, all_other_agents_types, lanes_mid, lanes, embedding_weight):
    del type, lanes_mid, lanes
    aoat = all_other_agents_types.astype(jnp.int32)
    aoat_flat = jnp.pad(
        aoat, ((0, 0), (0, AOAT_PAD - OTHER_LEN))).reshape(-1)
    w_flat = embedding_weight.reshape(-1)
    return _sc_call(aoat_flat, w_flat)
